# Initial kernel scaffold; baseline (speedup 1.0000x reference)
import jax
import jax.numpy as jnp
from jax.experimental import pallas as pl


def kernel(hs, hs_e, edge_index, W, b):
    raise NotImplementedError("write your pallas kernel here")


# speedup vs baseline: 3.0891x; 3.0891x over previous
import functools

import jax
import jax.numpy as jnp
from jax import lax
from jax.experimental import pallas as pl
from jax.experimental.pallas import tpu as pltpu
from jax.experimental.pallas import tpu_sc as plsc

N = 10000
E = 160000
D = 256

NC = 2
NS = 16
NW = NC * NS
CHUNK = 128
NCHUNKS = E // CHUNK
FLAG_W = 128
NP = 10240
ZROWS = NP // NS
ZBLK = 128


def _sc_gather_body(hs_hbm, dst_hbm, src_hbm, gdst_hbm, gsrc_hbm,
                    flags0_hbm, flags1_hbm,
                    idx_d, idx_s, buf_d, buf_s, ones_v, zbuf,
                    sem0, sem1, sem2):
  c = lax.axis_index("c")
  s = lax.axis_index("s")
  wid = c * NS + s

  def fill_ones(i, _):
    ones_v[i, :] = jnp.full((FLAG_W,), 1.0, dtype=jnp.float32)
    return 0
  lax.fori_loop(0, CHUNK, fill_ones, 0)

  def fill_zeros(i, _):
    zbuf[i, :] = jnp.zeros((FLAG_W,), dtype=jnp.float32)
    return 0
  lax.fori_loop(0, ZBLK, fill_zeros, 0)

  def zero_blk(j, _):
    @pl.when(c == 0)
    def _():
      pltpu.sync_copy(zbuf, flags0_hbm.at[pl.ds(s * ZROWS + j * ZBLK, ZBLK)])

    @pl.when(c == 1)
    def _():
      pltpu.sync_copy(zbuf, flags1_hbm.at[pl.ds(s * ZROWS + j * ZBLK, ZBLK)])
    return 0
  lax.fori_loop(0, ZROWS // ZBLK, zero_blk, 0)

  plsc.subcore_barrier()

  nchunks_w = (NCHUNKS + NW - 1 - wid) // NW

  def chunk_body(i, _):
    base = (wid + i * NW) * CHUNK
    pltpu.sync_copy(dst_hbm.at[pl.ds(base, CHUNK)], idx_d)
    pltpu.sync_copy(src_hbm.at[pl.ds(base, CHUNK)], idx_s)
    gd = pltpu.async_copy(hs_hbm.at[idx_d], buf_d, sem0)
    gs = pltpu.async_copy(hs_hbm.at[idx_s], buf_s, sem1)
    gd.wait()
    gs.wait()
    pltpu.sync_copy(buf_d, gdst_hbm.at[pl.ds(base, CHUNK)])
    pltpu.sync_copy(buf_s, gsrc_hbm.at[pl.ds(base, CHUNK)])

    @pl.when(c == 0)
    def _():
      pltpu.async_copy(ones_v, flags0_hbm.at[idx_d], sem2).wait()

    @pl.when(c == 1)
    def _():
      pltpu.async_copy(ones_v, flags1_hbm.at[idx_d], sem2).wait()

    return 0

  lax.fori_loop(0, nchunks_w, chunk_body, 0)


@jax.jit
def _sc_gather(hs, dst, src):
  mesh = plsc.VectorSubcoreMesh(core_axis_name="c", subcore_axis_name="s")
  return pl.kernel(
      _sc_gather_body,
      out_type=(
          jax.ShapeDtypeStruct((E, D), jnp.float32),
          jax.ShapeDtypeStruct((E, D), jnp.float32),
          jax.ShapeDtypeStruct((NP, FLAG_W), jnp.float32),
          jax.ShapeDtypeStruct((NP, FLAG_W), jnp.float32),
      ),
      mesh=mesh,
      scratch_types=(
          pltpu.VMEM((CHUNK,), jnp.int32),
          pltpu.VMEM((CHUNK,), jnp.int32),
          pltpu.VMEM((CHUNK, D), jnp.float32),
          pltpu.VMEM((CHUNK, D), jnp.float32),
          pltpu.VMEM((CHUNK, FLAG_W), jnp.float32),
          pltpu.VMEM((ZBLK, FLAG_W), jnp.float32),
          pltpu.SemaphoreType.DMA,
          pltpu.SemaphoreType.DMA,
          pltpu.SemaphoreType.DMA,
      ),
  )(hs, dst, src)


BE = 1280
BN = 80
GRID = E // BE


def _tc_body(hs_e_ref, gd_ref, gs_ref, wt_ref, b_ref, hs_ref,
             f0_ref, f1_ref, out_e_ref, out_n_ref):
  he = hs_e_ref[...]
  y = jnp.dot(he, wt_ref[...], preferred_element_type=jnp.float32)
  out_e_ref[...] = jax.nn.sigmoid(y + b_ref[...]) + gd_ref[...] + gs_ref[...] + he
  f = f0_ref[...] + f1_ref[...]
  m = jnp.sum(f, axis=1, keepdims=True) > 0.0
  out_n_ref[...] = jnp.where(m, hs_ref[...], 0.0)


@jax.jit
def _tc_fused(hs_e, gdst, gsrc, wt, b2, hs, flags0, flags1):
  return pl.pallas_call(
      _tc_body,
      grid=(GRID,),
      in_specs=[
          pl.BlockSpec((BE, D), lambda i: (i, 0)),
          pl.BlockSpec((BE, D), lambda i: (i, 0)),
          pl.BlockSpec((BE, D), lambda i: (i, 0)),
          pl.BlockSpec((D, D), lambda i: (0, 0)),
          pl.BlockSpec((1, D), lambda i: (0, 0)),
          pl.BlockSpec((BN, D), lambda i: (i, 0)),
          pl.BlockSpec((BN, FLAG_W), lambda i: (i, 0)),
          pl.BlockSpec((BN, FLAG_W), lambda i: (i, 0)),
      ],
      out_specs=[
          pl.BlockSpec((BE, D), lambda i: (i, 0)),
          pl.BlockSpec((BN, D), lambda i: (i, 0)),
      ],
      out_shape=[
          jax.ShapeDtypeStruct((E, D), jnp.float32),
          jax.ShapeDtypeStruct((N, D), jnp.float32),
      ],
      compiler_params=pltpu.CompilerParams(
          dimension_semantics=("arbitrary",),
      ),
  )(hs_e, gdst, gsrc, wt, b2, hs, flags0, flags1)


def kernel(hs, hs_e, edge_index, W, b):
  src = edge_index[0]
  dst = edge_index[1]
  gdst, gsrc, flags0, flags1 = _sc_gather(hs, dst, src)
  wt = W.T
  b2 = b.reshape(1, D)
  new_hs_e, useless = _tc_fused(hs_e, gdst, gsrc, wt, b2, hs, flags0, flags1)
  return new_hs_e, useless


# speedup vs baseline: 3.2185x; 1.0419x over previous
import functools

import jax
import jax.numpy as jnp
from jax import lax
from jax.experimental import pallas as pl
from jax.experimental.pallas import tpu as pltpu
from jax.experimental.pallas import tpu_sc as plsc

N = 10000
E = 160000
D = 256

NC = 2
NS = 16
NW = NC * NS
CHUNK = 128
NCHUNKS = E // CHUNK
NP = 10240
ZROWS = NP // NS


def _sc_gather_body(hs_hbm, dst_hbm, src_hbm, gdst_hbm, gsrc_hbm,
                    cnt0_hbm, cnt1_hbm,
                    idx_d, idx_s, buf_d, buf_s, ones_v, zbuf, counts_sh,
                    sem0, sem1):
  c = lax.axis_index("c")
  s = lax.axis_index("s")
  wid = c * NS + s

  def fill_ones(i, _):
    ones_v[pl.ds(i * 16, 16)] = jnp.full((16,), 1.0, dtype=jnp.float32)
    return 0
  lax.fori_loop(0, CHUNK // 16, fill_ones, 0)

  def fill_zeros(i, _):
    zbuf[pl.ds(i * 16, 16)] = jnp.zeros((16,), dtype=jnp.float32)
    return 0
  lax.fori_loop(0, ZROWS // 16, fill_zeros, 0)

  pltpu.sync_copy(zbuf, counts_sh.at[pl.ds(s * ZROWS, ZROWS)])
  plsc.subcore_barrier()

  nchunks_w = (NCHUNKS + NW - 1 - wid) // NW

  def chunk_body(i, _):
    base = (wid + i * NW) * CHUNK
    pltpu.sync_copy(dst_hbm.at[pl.ds(base, CHUNK)], idx_d)
    pltpu.sync_copy(src_hbm.at[pl.ds(base, CHUNK)], idx_s)
    gd = pltpu.async_copy(hs_hbm.at[idx_d], buf_d, sem0)
    gs = pltpu.async_copy(hs_hbm.at[idx_s], buf_s, sem1)
    pltpu.sync_copy(ones_v, counts_sh.at[idx_d], add=True)
    gd.wait()
    gs.wait()
    pltpu.sync_copy(buf_d, gdst_hbm.at[pl.ds(base, CHUNK)])
    pltpu.sync_copy(buf_s, gsrc_hbm.at[pl.ds(base, CHUNK)])
    return 0

  lax.fori_loop(0, nchunks_w, chunk_body, 0)

  plsc.subcore_barrier()

  @pl.when((s == 0) & (c == 0))
  def _():
    pltpu.sync_copy(counts_sh, cnt0_hbm)

  @pl.when((s == 0) & (c == 1))
  def _():
    pltpu.sync_copy(counts_sh, cnt1_hbm)


@jax.jit
def _sc_gather(hs, dst, src):
  mesh = plsc.VectorSubcoreMesh(core_axis_name="c", subcore_axis_name="s")
  return pl.kernel(
      _sc_gather_body,
      out_type=(
          jax.ShapeDtypeStruct((E, D), jnp.float32),
          jax.ShapeDtypeStruct((E, D), jnp.float32),
          jax.ShapeDtypeStruct((NP,), jnp.float32),
          jax.ShapeDtypeStruct((NP,), jnp.float32),
      ),
      mesh=mesh,
      scratch_types=(
          pltpu.VMEM((CHUNK,), jnp.int32),
          pltpu.VMEM((CHUNK,), jnp.int32),
          pltpu.VMEM((CHUNK, D), jnp.float32),
          pltpu.VMEM((CHUNK, D), jnp.float32),
          pltpu.VMEM((CHUNK,), jnp.float32),
          pltpu.VMEM((ZROWS,), jnp.float32),
          pltpu.VMEM_SHARED((NP,), jnp.float32),
          pltpu.SemaphoreType.DMA,
          pltpu.SemaphoreType.DMA,
      ),
  )(hs, dst, src)


BE = 1280
BN = 80
GRID = E // BE


def _tc_body(hs_e_ref, gd_ref, gs_ref, wt_ref, b_ref, hs_ref,
             c0_ref, c1_ref, out_e_ref, out_n_ref):
  he = hs_e_ref[...]
  y = jnp.dot(he, wt_ref[...], preferred_element_type=jnp.float32)
  out_e_ref[...] = jax.nn.sigmoid(y + b_ref[...]) + gd_ref[...] + gs_ref[...] + he
  m = (c0_ref[...] + c1_ref[...]) > 0.0
  out_n_ref[...] = jnp.where(m, hs_ref[...], 0.0)


@jax.jit
def _tc_fused(hs_e, gdst, gsrc, wt, b2, hs, cnt0, cnt1):
  return pl.pallas_call(
      _tc_body,
      grid=(GRID,),
      in_specs=[
          pl.BlockSpec((BE, D), lambda i: (i, 0)),
          pl.BlockSpec((BE, D), lambda i: (i, 0)),
          pl.BlockSpec((BE, D), lambda i: (i, 0)),
          pl.BlockSpec((D, D), lambda i: (0, 0)),
          pl.BlockSpec((1, D), lambda i: (0, 0)),
          pl.BlockSpec((BN, D), lambda i: (i, 0)),
          pl.BlockSpec((BN, 1), lambda i: (i, 0)),
          pl.BlockSpec((BN, 1), lambda i: (i, 0)),
      ],
      out_specs=[
          pl.BlockSpec((BE, D), lambda i: (i, 0)),
          pl.BlockSpec((BN, D), lambda i: (i, 0)),
      ],
      out_shape=[
          jax.ShapeDtypeStruct((E, D), jnp.float32),
          jax.ShapeDtypeStruct((N, D), jnp.float32),
      ],
      compiler_params=pltpu.CompilerParams(
          dimension_semantics=("arbitrary",),
      ),
  )(hs_e, gdst, gsrc, wt, b2, hs, cnt0, cnt1)


def kernel(hs, hs_e, edge_index, W, b):
  src = edge_index[0]
  dst = edge_index[1]
  gdst, gsrc, cnt0, cnt1 = _sc_gather(hs, dst, src)
  wt = W.T
  b2 = b.reshape(1, D)
  new_hs_e, useless = _tc_fused(
      hs_e, gdst, gsrc, wt, b2, hs, cnt0.reshape(NP, 1), cnt1.reshape(NP, 1))
  return new_hs_e, useless


# speedup vs baseline: 4.1369x; 1.2854x over previous
import functools

import jax
import jax.numpy as jnp
from jax import lax
from jax.experimental import pallas as pl
from jax.experimental.pallas import tpu as pltpu
from jax.experimental.pallas import tpu_sc as plsc

N = 10000
E = 160000
D = 256

NC = 2
NS = 16
NW = NC * NS
CHUNK = 64
NCHUNKS = E // CHUNK
NSETS = 3
NP = 10240
ZROWS = NP // NS


def _sc_gather_body(hs_hbm, dst_hbm, src_hbm, gsum_hbm, cnt0_hbm, cnt1_hbm,
                    idx_d0, idx_d1, idx_d2, idx_s0, idx_s1, idx_s2,
                    buf_d0, buf_d1, buf_d2, buf_s0, buf_s1, buf_s2,
                    ones_v, zbuf, counts_sh,
                    gd0, gd1, gd2, gs0, gs1, gs2,
                    w0, w1, w2, ss0, ss1, ss2):
  idx_d = (idx_d0, idx_d1, idx_d2)
  idx_s = (idx_s0, idx_s1, idx_s2)
  buf_d = (buf_d0, buf_d1, buf_d2)
  buf_s = (buf_s0, buf_s1, buf_s2)
  gdsem = (gd0, gd1, gd2)
  gssem = (gs0, gs1, gs2)
  wsem = (w0, w1, w2)
  ssem = (ss0, ss1, ss2)

  c = lax.axis_index("c")
  s = lax.axis_index("s")
  wid = c * NS + s
  n = (NCHUNKS + NW - 1 - wid) // NW

  def fill_ones(i, _):
    ones_v[pl.ds(i * 16, 16)] = jnp.full((16,), 1.0, dtype=jnp.float32)
    return 0
  lax.fori_loop(0, CHUNK // 16, fill_ones, 0)

  def fill_zeros(i, _):
    zbuf[pl.ds(i * 16, 16)] = jnp.zeros((16,), dtype=jnp.float32)
    return 0
  lax.fori_loop(0, ZROWS // 16, fill_zeros, 0)

  pltpu.sync_copy(zbuf, counts_sh.at[pl.ds(s * ZROWS, ZROWS)])
  plsc.subcore_barrier()

  def cbase(k):
    return (wid + k * NW) * CHUNK

  def issue(k, q):
    pltpu.sync_copy(dst_hbm.at[pl.ds(cbase(k), CHUNK)], idx_d[q])
    pltpu.sync_copy(src_hbm.at[pl.ds(cbase(k), CHUNK)], idx_s[q])
    pltpu.async_copy(hs_hbm.at[idx_d[q]], buf_d[q], gdsem[q])
    pltpu.async_copy(hs_hbm.at[idx_s[q]], buf_s[q], gssem[q])

    @pl.when(k < n - NSETS)
    def _():
      pltpu.async_copy(ones_v, counts_sh.at[idx_d[q]], ssem[q], add=True)

    @pl.when(k >= n - NSETS)
    def _():
      pltpu.sync_copy(ones_v, counts_sh.at[idx_d[q]], add=True)

  def substep(k, p):
    q = (p + 1) % NSETS

    @pl.when(k < n)
    def _():
      @pl.when(k + 1 < n)
      def _():
        @pl.when(k >= 2)
        def _():
          pltpu.make_async_copy(
              buf_d[q], gsum_hbm.at[pl.ds(0, CHUNK)], wsem[q]).wait()
          pltpu.make_async_copy(
              cnt0_hbm.at[pl.ds(0, CHUNK)], ones_v, ssem[q]).wait()
        issue(k + 1, q)

      pltpu.make_async_copy(hs_hbm.at[idx_d[p]], buf_d[p], gdsem[p]).wait()
      pltpu.make_async_copy(hs_hbm.at[idx_s[p]], buf_s[p], gssem[p]).wait()

      def row_add(r, _):
        for j in range(D // 16):
          sl = pl.ds(j * 16, 16)
          buf_d[p][r, sl] = buf_d[p][r, sl] + buf_s[p][r, sl]
        return 0
      lax.fori_loop(0, CHUNK, row_add, 0)

      @pl.when(k < n - NSETS)
      def _():
        pltpu.async_copy(buf_d[p], gsum_hbm.at[pl.ds(cbase(k), CHUNK)], wsem[p])

      @pl.when(k >= n - NSETS)
      def _():
        pltpu.sync_copy(buf_d[p], gsum_hbm.at[pl.ds(cbase(k), CHUNK)])

  issue(0, 0)

  def ring_body(g, _):
    substep(3 * g, 0)
    substep(3 * g + 1, 1)
    substep(3 * g + 2, 2)
    return 0
  lax.fori_loop(0, (n + NSETS - 1) // NSETS, ring_body, 0)

  plsc.subcore_barrier()

  @pl.when((s == 0) & (c == 0))
  def _():
    pltpu.sync_copy(counts_sh, cnt0_hbm)

  @pl.when((s == 0) & (c == 1))
  def _():
    pltpu.sync_copy(counts_sh, cnt1_hbm)


@jax.jit
def _sc_gather(hs, dst, src):
  mesh = plsc.VectorSubcoreMesh(core_axis_name="c", subcore_axis_name="s")
  return pl.kernel(
      _sc_gather_body,
      out_type=(
          jax.ShapeDtypeStruct((E, D), jnp.float32),
          jax.ShapeDtypeStruct((NP,), jnp.float32),
          jax.ShapeDtypeStruct((NP,), jnp.float32),
      ),
      mesh=mesh,
      scratch_types=(
          pltpu.VMEM((CHUNK,), jnp.int32),
          pltpu.VMEM((CHUNK,), jnp.int32),
          pltpu.VMEM((CHUNK,), jnp.int32),
          pltpu.VMEM((CHUNK,), jnp.int32),
          pltpu.VMEM((CHUNK,), jnp.int32),
          pltpu.VMEM((CHUNK,), jnp.int32),
          pltpu.VMEM((CHUNK, D), jnp.float32),
          pltpu.VMEM((CHUNK, D), jnp.float32),
          pltpu.VMEM((CHUNK, D), jnp.float32),
          pltpu.VMEM((CHUNK, D), jnp.float32),
          pltpu.VMEM((CHUNK, D), jnp.float32),
          pltpu.VMEM((CHUNK, D), jnp.float32),
          pltpu.VMEM((CHUNK,), jnp.float32),
          pltpu.VMEM((ZROWS,), jnp.float32),
          pltpu.VMEM_SHARED((NP,), jnp.float32),
          pltpu.SemaphoreType.DMA,
          pltpu.SemaphoreType.DMA,
          pltpu.SemaphoreType.DMA,
          pltpu.SemaphoreType.DMA,
          pltpu.SemaphoreType.DMA,
          pltpu.SemaphoreType.DMA,
          pltpu.SemaphoreType.DMA,
          pltpu.SemaphoreType.DMA,
          pltpu.SemaphoreType.DMA,
          pltpu.SemaphoreType.DMA,
          pltpu.SemaphoreType.DMA,
          pltpu.SemaphoreType.DMA,
      ),
  )(hs, dst, src)


BE = 1280
BN = 80
GRID = E // BE


def _tc_body(hs_e_ref, gsum_ref, wt_ref, b_ref, hs_ref,
             c0_ref, c1_ref, out_e_ref, out_n_ref):
  he = hs_e_ref[...]
  y = jnp.dot(he, wt_ref[...], preferred_element_type=jnp.float32)
  out_e_ref[...] = jax.nn.sigmoid(y + b_ref[...]) + gsum_ref[...] + he
  m = (c0_ref[...] + c1_ref[...]) > 0.0
  out_n_ref[...] = jnp.where(m, hs_ref[...], 0.0)


@jax.jit
def _tc_fused(hs_e, gsum, wt, b2, hs, cnt0, cnt1):
  return pl.pallas_call(
      _tc_body,
      grid=(GRID,),
      in_specs=[
          pl.BlockSpec((BE, D), lambda i: (i, 0)),
          pl.BlockSpec((BE, D), lambda i: (i, 0)),
          pl.BlockSpec((D, D), lambda i: (0, 0)),
          pl.BlockSpec((1, D), lambda i: (0, 0)),
          pl.BlockSpec((BN, D), lambda i: (i, 0)),
          pl.BlockSpec((BN, 1), lambda i: (i, 0)),
          pl.BlockSpec((BN, 1), lambda i: (i, 0)),
      ],
      out_specs=[
          pl.BlockSpec((BE, D), lambda i: (i, 0)),
          pl.BlockSpec((BN, D), lambda i: (i, 0)),
      ],
      out_shape=[
          jax.ShapeDtypeStruct((E, D), jnp.float32),
          jax.ShapeDtypeStruct((N, D), jnp.float32),
      ],
      compiler_params=pltpu.CompilerParams(
          dimension_semantics=("arbitrary",),
      ),
  )(hs_e, gsum, wt, b2, hs, cnt0, cnt1)


def kernel(hs, hs_e, edge_index, W, b):
  src = edge_index[0]
  dst = edge_index[1]
  gsum, cnt0, cnt1 = _sc_gather(hs, dst, src)
  wt = W.T
  b2 = b.reshape(1, D)
  new_hs_e, useless = _tc_fused(
      hs_e, gsum, wt, b2, hs, cnt0.reshape(NP, 1), cnt1.reshape(NP, 1))
  return new_hs_e, useless


# speedup vs baseline: 4.4686x; 1.0802x over previous
import functools

import jax
import jax.numpy as jnp
from jax import lax
from jax.experimental import pallas as pl
from jax.experimental.pallas import tpu as pltpu
from jax.experimental.pallas import tpu_sc as plsc

N = 10000
E = 160000
D = 256

NC = 2
NS = 16
NW = NC * NS
CHUNK = 64
NCHUNKS = E // CHUNK
NSETS = 3
NP = 10240
ZROWS = NP // NS


def _sc_gather_body(hs_hbm, dst_hbm, src_hbm, gsum_hbm, cnt0_hbm, cnt1_hbm,
                    idx_d0, idx_d1, idx_d2, idx_s0, idx_s1, idx_s2,
                    buf_d0, buf_d1, buf_d2, buf_s0, buf_s1, buf_s2,
                    ones_v, zbuf, counts_sh,
                    gd0, gd1, gd2, gs0, gs1, gs2,
                    w0, w1, w2, ss0, ss1, ss2):
  idx_d = (idx_d0, idx_d1, idx_d2)
  idx_s = (idx_s0, idx_s1, idx_s2)
  buf_d = (buf_d0, buf_d1, buf_d2)
  buf_s = (buf_s0, buf_s1, buf_s2)
  gdsem = (gd0, gd1, gd2)
  gssem = (gs0, gs1, gs2)
  wsem = (w0, w1, w2)
  ssem = (ss0, ss1, ss2)

  c = lax.axis_index("c")
  s = lax.axis_index("s")
  wid = c * NS + s
  n = (NCHUNKS + NW - 1 - wid) // NW

  def fill_ones(i, _):
    ones_v[pl.ds(i * 16, 16)] = jnp.full((16,), 1.0, dtype=jnp.float32)
    return 0
  lax.fori_loop(0, CHUNK // 16, fill_ones, 0)

  def fill_zeros(i, _):
    zbuf[pl.ds(i * 16, 16)] = jnp.zeros((16,), dtype=jnp.float32)
    return 0
  lax.fori_loop(0, ZROWS // 16, fill_zeros, 0)

  pltpu.sync_copy(zbuf, counts_sh.at[pl.ds(s * ZROWS, ZROWS)])
  plsc.subcore_barrier()

  def cbase(k):
    return (wid + k * NW) * CHUNK

  def issue(k, q):
    pltpu.sync_copy(dst_hbm.at[pl.ds(cbase(k), CHUNK)], idx_d[q])
    pltpu.sync_copy(src_hbm.at[pl.ds(cbase(k), CHUNK)], idx_s[q])
    pltpu.async_copy(hs_hbm.at[idx_d[q]], buf_d[q], gdsem[q])
    pltpu.async_copy(hs_hbm.at[idx_s[q]], buf_s[q], gssem[q])

    @pl.when(k < n - NSETS)
    def _():
      pltpu.async_copy(ones_v, counts_sh.at[idx_d[q]], ssem[q], add=True)

    @pl.when(k >= n - NSETS)
    def _():
      pltpu.sync_copy(ones_v, counts_sh.at[idx_d[q]], add=True)

  def substep(k, p):
    q = (p + 1) % NSETS

    @pl.when(k < n)
    def _():
      @pl.when(k + 1 < n)
      def _():
        @pl.when(k >= 2)
        def _():
          pltpu.make_async_copy(
              buf_d[q], gsum_hbm.at[pl.ds(0, CHUNK)], wsem[q]).wait()
          pltpu.make_async_copy(
              cnt0_hbm.at[pl.ds(0, CHUNK)], ones_v, ssem[q]).wait()
        issue(k + 1, q)

      pltpu.make_async_copy(hs_hbm.at[idx_d[p]], buf_d[p], gdsem[p]).wait()
      pltpu.make_async_copy(hs_hbm.at[idx_s[p]], buf_s[p], gssem[p]).wait()

      def row_add(r, _):
        for j in range(D // 16):
          sl = pl.ds(j * 16, 16)
          buf_d[p][r, sl] = buf_d[p][r, sl] + buf_s[p][r, sl]
        return 0
      lax.fori_loop(0, CHUNK, row_add, 0)

      @pl.when(k < n - NSETS)
      def _():
        pltpu.async_copy(buf_d[p], gsum_hbm.at[pl.ds(cbase(k), CHUNK)], wsem[p])

      @pl.when(k >= n - NSETS)
      def _():
        pltpu.sync_copy(buf_d[p], gsum_hbm.at[pl.ds(cbase(k), CHUNK)])

  issue(0, 0)

  def ring_body(g, _):
    substep(3 * g, 0)
    substep(3 * g + 1, 1)
    substep(3 * g + 2, 2)
    return 0
  lax.fori_loop(0, (n + NSETS - 1) // NSETS, ring_body, 0)

  plsc.subcore_barrier()

  @pl.when((s == 0) & (c == 0))
  def _():
    pltpu.sync_copy(counts_sh, cnt0_hbm)

  @pl.when((s == 0) & (c == 1))
  def _():
    pltpu.sync_copy(counts_sh, cnt1_hbm)


@jax.jit
def _sc_gather(hs, dst, src):
  mesh = plsc.VectorSubcoreMesh(core_axis_name="c", subcore_axis_name="s")
  return pl.kernel(
      _sc_gather_body,
      out_type=(
          jax.ShapeDtypeStruct((E, D), jnp.float32),
          jax.ShapeDtypeStruct((NP,), jnp.float32),
          jax.ShapeDtypeStruct((NP,), jnp.float32),
      ),
      mesh=mesh,
      scratch_types=(
          pltpu.VMEM((CHUNK,), jnp.int32),
          pltpu.VMEM((CHUNK,), jnp.int32),
          pltpu.VMEM((CHUNK,), jnp.int32),
          pltpu.VMEM((CHUNK,), jnp.int32),
          pltpu.VMEM((CHUNK,), jnp.int32),
          pltpu.VMEM((CHUNK,), jnp.int32),
          pltpu.VMEM((CHUNK, D), jnp.float32),
          pltpu.VMEM((CHUNK, D), jnp.float32),
          pltpu.VMEM((CHUNK, D), jnp.float32),
          pltpu.VMEM((CHUNK, D), jnp.float32),
          pltpu.VMEM((CHUNK, D), jnp.float32),
          pltpu.VMEM((CHUNK, D), jnp.float32),
          pltpu.VMEM((CHUNK,), jnp.float32),
          pltpu.VMEM((ZROWS,), jnp.float32),
          pltpu.VMEM_SHARED((NP,), jnp.float32),
          pltpu.SemaphoreType.DMA,
          pltpu.SemaphoreType.DMA,
          pltpu.SemaphoreType.DMA,
          pltpu.SemaphoreType.DMA,
          pltpu.SemaphoreType.DMA,
          pltpu.SemaphoreType.DMA,
          pltpu.SemaphoreType.DMA,
          pltpu.SemaphoreType.DMA,
          pltpu.SemaphoreType.DMA,
          pltpu.SemaphoreType.DMA,
          pltpu.SemaphoreType.DMA,
          pltpu.SemaphoreType.DMA,
      ),
  )(hs, dst, src)


BE = 3200
BN = 200
GRID = E // BE


def _tc_body(hs_e_ref, gsum_ref, wt_ref, b_ref, hs_ref,
             c0_ref, c1_ref, out_e_ref, out_n_ref):
  he = hs_e_ref[...]
  y = jnp.dot(he.astype(jnp.bfloat16), wt_ref[...],
              preferred_element_type=jnp.float32)
  out_e_ref[...] = jax.nn.sigmoid(y + b_ref[...]) + gsum_ref[...] + he
  m = (c0_ref[...] + c1_ref[...]) > 0.0
  out_n_ref[...] = jnp.where(m, hs_ref[...], 0.0)


@jax.jit
def _tc_fused(hs_e, gsum, wt, b2, hs, cnt0, cnt1):
  return pl.pallas_call(
      _tc_body,
      grid=(GRID,),
      in_specs=[
          pl.BlockSpec((BE, D), lambda i: (i, 0)),
          pl.BlockSpec((BE, D), lambda i: (i, 0)),
          pl.BlockSpec((D, D), lambda i: (0, 0)),
          pl.BlockSpec((1, D), lambda i: (0, 0)),
          pl.BlockSpec((BN, D), lambda i: (i, 0)),
          pl.BlockSpec((BN, 1), lambda i: (i, 0)),
          pl.BlockSpec((BN, 1), lambda i: (i, 0)),
      ],
      out_specs=[
          pl.BlockSpec((BE, D), lambda i: (i, 0)),
          pl.BlockSpec((BN, D), lambda i: (i, 0)),
      ],
      out_shape=[
          jax.ShapeDtypeStruct((E, D), jnp.float32),
          jax.ShapeDtypeStruct((N, D), jnp.float32),
      ],
      compiler_params=pltpu.CompilerParams(
          dimension_semantics=("arbitrary",),
      ),
  )(hs_e, gsum, wt, b2, hs, cnt0, cnt1)


def kernel(hs, hs_e, edge_index, W, b):
  src = edge_index[0]
  dst = edge_index[1]
  gsum, cnt0, cnt1 = _sc_gather(hs, dst, src)
  wt = W.T.astype(jnp.bfloat16)
  b2 = b.reshape(1, D)
  new_hs_e, useless = _tc_fused(
      hs_e, gsum, wt, b2, hs, cnt0.reshape(NP, 1), cnt1.reshape(NP, 1))
  return new_hs_e, useless
